# baseline (speedup 1.0000x reference)
import functools

import jax
import jax.numpy as jnp
from jax import lax
from jax.experimental import pallas as pl
from jax.experimental.pallas import tpu as pltpu
from jax.experimental.pallas import tpu_sc as plsc

_NC = 2
_NS = 16
_NW = _NC * _NS
_L = 16
_B_SUB = 128
_G = 4


def _encoder_body(D, H, b_per_w, n_chunks,
                  tags_p, cats, tag_packed, cat_table,
                  out_mixed, out_cats,
                  idx0, idx1, cidx_v, gbuf0, gbuf1, obuf0, obuf1, cbuf,
                  sem0, sem1, csem, isem0, isem1, osem0, osem1):
    nc = (D // 2) // _L
    rows = _G * H
    npc = _B_SUB // _G
    wid = lax.axis_index("s") * _NC + lax.axis_index("c")
    base = wid * b_per_w

    idxs = (idx0, idx1)
    isems = (isem0, isem1)
    obufs = (obuf0, obuf1)
    osems = (osem0, osem1)
    bufs = (gbuf0, gbuf1)
    sems = (sem0, sem1)

    def fire(idx_ref, p, u):
        pltpu.async_copy(
            tag_packed.at[idx_ref.at[pl.ds(p * rows, rows)]], bufs[u], sems[u])

    def wait_buf(u):
        pltpu.make_async_copy(
            tag_packed.at[pl.ds(0, rows)], bufs[u], sems[u]).wait()

    def idx_start(ch, ph):
        cb = base + ch * _B_SUB
        pltpu.async_copy(
            tags_p.at[pl.ds(cb * H, _B_SUB * H)], idxs[ph], isems[ph])

    def idx_wait(ph):
        pltpu.make_async_copy(
            tags_p.at[pl.ds(0, _B_SUB * H)], idxs[ph], isems[ph]).wait()

    def obuf_drain_start(ch, ph):
        cb = base + ch * _B_SUB
        pltpu.async_copy(obufs[ph], out_mixed.at[pl.ds(cb, _B_SUB)], osems[ph])

    def obuf_drain_wait(ph):
        pltpu.make_async_copy(
            obufs[ph], out_mixed.at[pl.ds(0, _B_SUB)], osems[ph]).wait()

    def accum(ph, p, u):
        buf = bufs[u]
        zero = jnp.zeros((_L,), jnp.float32)
        for e in range(_G):
            off = e * H

            def add_row(h, carry):
                acc = list(carry)
                for c in range(nc):
                    v = buf[off + h, pl.ds(c * _L, _L)]
                    lo = plsc.bitcast(v << 16, jnp.float32)
                    hi = plsc.bitcast(v, jnp.float32)
                    acc[2 * c] = acc[2 * c] + lo
                    acc[2 * c + 1] = acc[2 * c + 1] + hi
                return tuple(acc)

            acc = lax.fori_loop(0, H, add_row, (zero,) * (2 * nc), unroll=8)
            for j in range(2 * nc):
                obufs[ph][_G * p + e, pl.ds(j * _L, _L)] = acc[j]

    pltpu.sync_copy(tags_p.at[pl.ds(base * H, _B_SUB * H)], idx0)
    idx_start(1, 1)
    fire(idx0, 0, 0)

    def chunk_pair(ci, carry):
        for ph in range(2):
            ch = 2 * ci + ph
            cb = base + ch * _B_SUB

            @pl.when(ch >= 2)
            def _():
                obuf_drain_wait(ph)

            pltpu.sync_copy(cats.at[pl.ds(cb, _B_SUB)], cidx_v)
            pltpu.async_copy(cat_table.at[cidx_v], cbuf, csem)

            def group_body(i, c2):
                for u in range(2):
                    g = 2 * i + u

                    @pl.when(g + 1 < npc)
                    def _():
                        fire(idxs[ph], g + 1, (u + 1) % 2)

                    @pl.when((g + 1 == npc) & (ch + 1 < n_chunks))
                    def _():
                        idx_wait(1 - ph)
                        fire(idxs[1 - ph], 0, (u + 1) % 2)

                    wait_buf(u)
                    accum(ph, g, u)
                return c2

            lax.fori_loop(0, npc // 2, group_body, 0)
            obuf_drain_start(ch, ph)
            pltpu.make_async_copy(
                cat_table.at[pl.ds(0, _B_SUB)], cbuf, csem).wait()
            pltpu.sync_copy(cbuf, out_cats.at[pl.ds(cb, _B_SUB)])

            @pl.when(ch + 2 < n_chunks)
            def _():
                idx_start(ch + 2, ph)

        return carry

    lax.fori_loop(0, n_chunks // 2, chunk_pair, 0)
    obuf_drain_wait(0)
    obuf_drain_wait(1)


def kernel(tags, categories, tag_table, cat_table):
    H, B = tags.shape
    V, D = tag_table.shape
    b_per_w = B // _NW
    n_chunks = b_per_w // _B_SUB
    assert n_chunks >= 2 and n_chunks % 2 == 0

    tags_p = tags.T.reshape(-1)

    tag_packed = jax.lax.bitcast_convert_type(
        tag_table.astype(jnp.bfloat16).reshape(V, D // 2, 2), jnp.int32)

    mesh = plsc.VectorSubcoreMesh(
        core_axis_name="c", subcore_axis_name="s",
        num_cores=_NC, num_subcores=_NS)
    f = pl.kernel(
        functools.partial(_encoder_body, D, H, b_per_w, n_chunks),
        out_type=(jax.ShapeDtypeStruct((B, D), jnp.float32),
                  jax.ShapeDtypeStruct((B, D), jnp.float32)),
        mesh=mesh,
        compiler_params=pltpu.CompilerParams(
            use_tc_tiling_on_sc=False, needs_layout_passes=False),
        scratch_types=[
            pltpu.VMEM((_B_SUB * H,), jnp.int32),
            pltpu.VMEM((_B_SUB * H,), jnp.int32),
            pltpu.VMEM((_B_SUB,), jnp.int32),
            pltpu.VMEM((_G * H, D // 2), jnp.int32),
            pltpu.VMEM((_G * H, D // 2), jnp.int32),
            pltpu.VMEM((_B_SUB, D), jnp.float32),
            pltpu.VMEM((_B_SUB, D), jnp.float32),
            pltpu.VMEM((_B_SUB, D), jnp.float32),
            pltpu.SemaphoreType.DMA,
            pltpu.SemaphoreType.DMA,
            pltpu.SemaphoreType.DMA,
            pltpu.SemaphoreType.DMA,
            pltpu.SemaphoreType.DMA,
            pltpu.SemaphoreType.DMA,
            pltpu.SemaphoreType.DMA,
        ],
    )
    out_mixed, out_cats = f(tags_p, categories, tag_packed, cat_table)
    out_tags = out_mixed.reshape(B, 2, 2, _L).transpose(0, 1, 3, 2).reshape(B, D)
    return (out_tags, out_cats)


# speedup vs baseline: 1.0949x; 1.0949x over previous
import functools

import jax
import jax.numpy as jnp
from jax import lax
from jax.experimental import pallas as pl
from jax.experimental.pallas import tpu as pltpu
from jax.experimental.pallas import tpu_sc as plsc

_NC = 2
_NS = 16
_NW = _NC * _NS
_L = 16
_B_SUB = 128
_G = 2


def _encoder_body(D, H, b_per_w, n_chunks,
                  tags_p, cats, tag_table, cat_table,
                  out_tags, out_cats,
                  idx0, idx1, cidx_v, gbuf0, gbuf1, obuf0, obuf1, cbuf,
                  sem0, sem1, csem, isem0, isem1, osem0, osem1):
    nd = D // _L
    rows = _G * H
    npc = _B_SUB // _G
    wid = lax.axis_index("s") * _NC + lax.axis_index("c")
    base = wid * b_per_w

    idxs = (idx0, idx1)
    isems = (isem0, isem1)
    obufs = (obuf0, obuf1)
    osems = (osem0, osem1)
    bufs = (gbuf0, gbuf1)
    sems = (sem0, sem1)

    def fire(idx_ref, p, u):
        pltpu.async_copy(
            tag_table.at[idx_ref.at[pl.ds(p * rows, rows)]], bufs[u], sems[u])

    def wait_buf(u):
        pltpu.make_async_copy(
            tag_table.at[pl.ds(0, rows)], bufs[u], sems[u]).wait()

    def idx_start(ch, ph):
        cb = base + ch * _B_SUB
        pltpu.async_copy(
            tags_p.at[pl.ds(cb * H, _B_SUB * H)], idxs[ph], isems[ph])

    def idx_wait(ph):
        pltpu.make_async_copy(
            tags_p.at[pl.ds(0, _B_SUB * H)], idxs[ph], isems[ph]).wait()

    def obuf_drain_start(ch, ph):
        cb = base + ch * _B_SUB
        pltpu.async_copy(obufs[ph], out_tags.at[pl.ds(cb, _B_SUB)], osems[ph])

    def obuf_drain_wait(ph):
        pltpu.make_async_copy(
            obufs[ph], out_tags.at[pl.ds(0, _B_SUB)], osems[ph]).wait()

    def accum(ph, p, u):
        buf = bufs[u]
        zero = jnp.zeros((_L,), jnp.float32)
        for e in range(_G):
            off = e * H

            def add_row(h, carry):
                return tuple(
                    carry[d] + buf[off + h, pl.ds(d * _L, _L)]
                    for d in range(nd))

            acc = lax.fori_loop(0, H, add_row, (zero,) * nd, unroll=8)
            for j in range(nd):
                obufs[ph][_G * p + e, pl.ds(j * _L, _L)] = acc[j]

    pltpu.sync_copy(tags_p.at[pl.ds(base * H, _B_SUB * H)], idx0)
    idx_start(1, 1)
    fire(idx0, 0, 0)

    def chunk_pair(ci, carry):
        for ph in range(2):
            ch = 2 * ci + ph
            cb = base + ch * _B_SUB

            @pl.when(ch >= 2)
            def _():
                obuf_drain_wait(ph)

            pltpu.sync_copy(cats.at[pl.ds(cb, _B_SUB)], cidx_v)
            pltpu.async_copy(cat_table.at[cidx_v], cbuf, csem)

            def group_body(i, c2):
                for u in range(2):
                    g = 2 * i + u

                    @pl.when(g + 1 < npc)
                    def _():
                        fire(idxs[ph], g + 1, (u + 1) % 2)

                    @pl.when((g + 1 == npc) & (ch + 1 < n_chunks))
                    def _():
                        idx_wait(1 - ph)
                        fire(idxs[1 - ph], 0, (u + 1) % 2)

                    wait_buf(u)
                    accum(ph, g, u)
                return c2

            lax.fori_loop(0, npc // 2, group_body, 0)
            obuf_drain_start(ch, ph)
            pltpu.make_async_copy(
                cat_table.at[pl.ds(0, _B_SUB)], cbuf, csem).wait()
            pltpu.sync_copy(cbuf, out_cats.at[pl.ds(cb, _B_SUB)])

            @pl.when(ch + 2 < n_chunks)
            def _():
                idx_start(ch + 2, ph)

        return carry

    lax.fori_loop(0, n_chunks // 2, chunk_pair, 0)
    obuf_drain_wait(0)
    obuf_drain_wait(1)


def kernel(tags, categories, tag_table, cat_table):
    H, B = tags.shape
    V, D = tag_table.shape
    b_per_w = B // _NW
    n_chunks = b_per_w // _B_SUB
    assert n_chunks >= 2 and n_chunks % 2 == 0

    tags_p = tags.T.reshape(-1)

    mesh = plsc.VectorSubcoreMesh(
        core_axis_name="c", subcore_axis_name="s",
        num_cores=_NC, num_subcores=_NS)
    f = pl.kernel(
        functools.partial(_encoder_body, D, H, b_per_w, n_chunks),
        out_type=(jax.ShapeDtypeStruct((B, D), jnp.float32),
                  jax.ShapeDtypeStruct((B, D), jnp.float32)),
        mesh=mesh,
        compiler_params=pltpu.CompilerParams(
            use_tc_tiling_on_sc=False, needs_layout_passes=False),
        scratch_types=[
            pltpu.VMEM((_B_SUB * H,), jnp.int32),
            pltpu.VMEM((_B_SUB * H,), jnp.int32),
            pltpu.VMEM((_B_SUB,), jnp.int32),
            pltpu.VMEM((_G * H, D), jnp.float32),
            pltpu.VMEM((_G * H, D), jnp.float32),
            pltpu.VMEM((_B_SUB, D), jnp.float32),
            pltpu.VMEM((_B_SUB, D), jnp.float32),
            pltpu.VMEM((_B_SUB, D), jnp.float32),
            pltpu.SemaphoreType.DMA,
            pltpu.SemaphoreType.DMA,
            pltpu.SemaphoreType.DMA,
            pltpu.SemaphoreType.DMA,
            pltpu.SemaphoreType.DMA,
            pltpu.SemaphoreType.DMA,
            pltpu.SemaphoreType.DMA,
        ],
    )
    return f(tags_p, categories, tag_table, cat_table)
